# baseline (speedup 1.0000x reference)
import functools

import jax
import jax.numpy as jnp
from jax import lax
from jax.experimental import pallas as pl
from jax.experimental.pallas import tpu as pltpu
from jax.experimental.pallas import tpu_sc as plsc

NC = 2
NS = 16
NW = NC * NS
CH = 128
NQ = 2
DW = 16


@functools.lru_cache(maxsize=None)
def _make_sc_scatter(N, F, Ep, with_deg):
    CPW = Ep // (NW * CH)
    iters = CPW // NQ
    Nz = ((N + NS * 8 - 1) // (NS * 8)) * (NS * 8)
    NPS = Nz // NS
    Nacc = Nz + 16

    mesh = plsc.VectorSubcoreMesh(
        core_axis_name="c", subcore_axis_name="s", num_cores=NC, num_subcores=NS
    )

    out_type = [jax.ShapeDtypeStruct((NC, Nz, F), jnp.float32)]
    scratch = [
        pltpu.VMEM((NQ * 2, CH), jnp.int32),
        pltpu.VMEM((NQ * CH, F), jnp.float32),
        pltpu.VMEM_SHARED((Nacc, F), jnp.float32),
        pltpu.SemaphoreType.DMA,
        pltpu.SemaphoreType.DMA,
    ]
    if with_deg:
        out_type.append(jax.ShapeDtypeStruct((NC, Nz, DW), jnp.float32))
        scratch += [
            pltpu.VMEM((CH, DW), jnp.float32),
            pltpu.VMEM_SHARED((Nacc, DW), jnp.float32),
            pltpu.SemaphoreType.DMA,
        ]

    def body(idx_hbm, h_hbm, zf_hbm, zd_hbm, e1_hbm, p_hbm, *rest):
        if with_deg:
            (dp_hbm, idx_v, rows, acc, sem_g, sem_s,
             ones_v, dacc, sem_d) = rest
        else:
            idx_v, rows, acc, sem_g, sem_s = rest
        cid = lax.axis_index("c")
        sid = lax.axis_index("s")
        wid = cid * NS + sid

        pltpu.sync_copy(zf_hbm, acc.at[pl.ds(sid * NPS, NPS)])
        if with_deg:
            pltpu.sync_copy(zd_hbm, dacc.at[pl.ds(sid * NPS, NPS)])
            pltpu.sync_copy(e1_hbm, ones_v)
        plsc.subcore_barrier()

        def step(i, carry):
            base = (wid * CPW + i * NQ) * 2
            pltpu.sync_copy(idx_hbm.at[pl.ds(base, NQ * 2)], idx_v)
            gats = [
                pltpu.async_copy(h_hbm.at[idx_v.at[2 * q + 1]],
                                 rows.at[pl.ds(q * CH, CH)], sem_g)
                for q in range(NQ)
            ]
            for g in gats:
                g.wait()
            scats = [
                pltpu.async_copy(rows.at[pl.ds(q * CH, CH)],
                                 acc.at[idx_v.at[2 * q]], sem_s, add=True)
                for q in range(NQ)
            ]
            degs = []
            if with_deg:
                degs = [
                    pltpu.async_copy(ones_v, dacc.at[idx_v.at[2 * q]], sem_d,
                                     add=True)
                    for q in range(NQ)
                ]
            for s in scats:
                s.wait()
            for s in degs:
                s.wait()
            return carry

        lax.fori_loop(0, iters, step, 0)
        plsc.subcore_barrier()

        sl = pl.ds(sid * NPS, NPS)
        pltpu.sync_copy(acc.at[sl], p_hbm.at[cid].at[sl])
        if with_deg:
            pltpu.sync_copy(dacc.at[sl], dp_hbm.at[cid].at[sl])

    return pl.kernel(body, out_type=tuple(out_type), mesh=mesh,
                     scratch_types=tuple(scratch),
                     compiler_params=pltpu.CompilerParams(
                         use_tc_tiling_on_sc=False))


def _edge_chunks(edge_index, N):
    E = edge_index.shape[1]
    Nz = ((N + NS * 8 - 1) // (NS * 8)) * (NS * 8)
    unit = NW * CH * NQ
    Ep = ((E + unit - 1) // unit) * unit
    if Ep != E:
        pad = jnp.stack([jnp.full((Ep - E,), Nz, jnp.int32),
                         jnp.zeros((Ep - E,), jnp.int32)])
        edge_index = jnp.concatenate([edge_index, pad], axis=1)
    chunks = edge_index.reshape(2, Ep // CH, CH).transpose(1, 0, 2)
    return chunks.reshape(Ep // CH * 2, CH), Ep


def _sc_scatter(idx_chunks, Ep, h, with_deg):
    N, F = h.shape
    Nz = ((N + NS * 8 - 1) // (NS * 8)) * (NS * 8)
    NPS = Nz // NS
    zf = jnp.zeros((NPS, F), jnp.float32)
    zd = jnp.zeros((NPS, DW), jnp.float32)
    e1 = jnp.zeros((CH, DW), jnp.float32).at[:, 0].set(1.0)
    k = _make_sc_scatter(N, F, Ep, with_deg)
    return k(idx_chunks, h, zf, zd, e1)


def _mm_body(x_ref, w_ref, o_ref):
    o_ref[...] = jnp.dot(x_ref[...], w_ref[...],
                         preferred_element_type=jnp.float32)


def _mid_body(p_ref, h1_ref, dp_ref, b1_ref, w2_ref, o_ref):
    deg = dp_ref[0, :, 0:1] + dp_ref[1, :, 0:1] + 1.0
    s = (p_ref[0] + p_ref[1] + h1_ref[...]) * (1.0 / deg) + b1_ref[...]
    h = jnp.maximum(s, 0.0)
    o_ref[...] = jnp.dot(h, w2_ref[...], preferred_element_type=jnp.float32)


def _out_body(q_ref, h2_ref, dp_ref, b2_ref, o_ref):
    deg = dp_ref[0, :, 0:1] + dp_ref[1, :, 0:1] + 1.0
    o_ref[...] = (q_ref[0] + q_ref[1] + h2_ref[...]) * (1.0 / deg) \
        + b2_ref[...]


def _row_block(N):
    for r in (1000, 500, 250, 200, 125, 100, 50, 40, 25, 20, 10, 8, 5, 4, 2):
        if N % r == 0:
            return r
    return N


def kernel(x, edge_index, W1, b1, W2, b2):
    N, NF = x.shape
    NH = W1.shape[1]
    F2 = W2.shape[1]
    idx_chunks, Ep = _edge_chunks(edge_index, N)
    R = _row_block(N)
    G = N // R

    h1 = pl.pallas_call(
        _mm_body,
        grid=(G,),
        in_specs=[pl.BlockSpec((R, NF), lambda i: (i, 0)),
                  pl.BlockSpec((NF, NH), lambda i: (0, 0))],
        out_specs=pl.BlockSpec((R, NH), lambda i: (i, 0)),
        out_shape=jax.ShapeDtypeStruct((N, NH), jnp.float32),
    )(x, W1)

    p1, dp = _sc_scatter(idx_chunks, Ep, h1, with_deg=True)

    h2 = pl.pallas_call(
        _mid_body,
        grid=(G,),
        in_specs=[pl.BlockSpec((NC, R, NH), lambda i: (0, i, 0)),
                  pl.BlockSpec((R, NH), lambda i: (i, 0)),
                  pl.BlockSpec((NC, R, DW), lambda i: (0, i, 0)),
                  pl.BlockSpec((1, NH), lambda i: (0, 0)),
                  pl.BlockSpec((NH, F2), lambda i: (0, 0))],
        out_specs=pl.BlockSpec((R, F2), lambda i: (i, 0)),
        out_shape=jax.ShapeDtypeStruct((N, F2), jnp.float32),
    )(p1[:, :N], h1, dp[:, :N], b1.reshape(1, NH), W2)

    (p2,) = _sc_scatter(idx_chunks, Ep, h2, with_deg=False)

    out = pl.pallas_call(
        _out_body,
        grid=(G,),
        in_specs=[pl.BlockSpec((NC, R, F2), lambda i: (0, i, 0)),
                  pl.BlockSpec((R, F2), lambda i: (i, 0)),
                  pl.BlockSpec((NC, R, DW), lambda i: (0, i, 0)),
                  pl.BlockSpec((1, F2), lambda i: (0, 0))],
        out_specs=pl.BlockSpec((R, F2), lambda i: (i, 0)),
        out_shape=jax.ShapeDtypeStruct((N, F2), jnp.float32),
    )(p2[:, :N], h2, dp[:, :N], b2.reshape(1, F2))

    return out


# speedup vs baseline: 1.1338x; 1.1338x over previous
import functools

import jax
import jax.numpy as jnp
from jax import lax
from jax.experimental import pallas as pl
from jax.experimental.pallas import tpu as pltpu
from jax.experimental.pallas import tpu_sc as plsc

NC = 2
NS = 16
NW = NC * NS
CH = 64
NQ = 4
DW = 16


@functools.lru_cache(maxsize=None)
def _make_sc_scatter(N, F, Ep, with_deg):
    CPW = Ep // (NW * CH)
    OUTER = CPW // NQ
    Nz = ((N + NS * 8 - 1) // (NS * 8)) * (NS * 8)
    NPS = Nz // NS
    Nacc = Nz + 16

    mesh = plsc.VectorSubcoreMesh(
        core_axis_name="c", subcore_axis_name="s", num_cores=NC, num_subcores=NS
    )

    out_type = [jax.ShapeDtypeStruct((NC, Nz, F), jnp.float32)]
    scratch = [
        pltpu.VMEM((NQ * 2, CH), jnp.int32),
        pltpu.VMEM((NQ * CH, F), jnp.float32),
        pltpu.VMEM_SHARED((Nacc, F), jnp.float32),
        [pltpu.SemaphoreType.DMA for _ in range(NQ)],
        [pltpu.SemaphoreType.DMA for _ in range(NQ)],
    ]
    if with_deg:
        out_type.append(jax.ShapeDtypeStruct((NC, Nz, DW), jnp.float32))
        scratch += [
            pltpu.VMEM((CH, DW), jnp.float32),
            pltpu.VMEM_SHARED((Nacc, DW), jnp.float32),
            [pltpu.SemaphoreType.DMA for _ in range(NQ)],
        ]

    def body(idx_hbm, h_hbm, zf_hbm, zd_hbm, e1_hbm, p_hbm, *rest):
        if with_deg:
            (dp_hbm, idx_v, rows, acc, sem_g, sem_s,
             ones_v, dacc, sem_d) = rest
        else:
            idx_v, rows, acc, sem_g, sem_s = rest
        cid = lax.axis_index("c")
        sid = lax.axis_index("s")
        wid = cid * NS + sid

        pltpu.sync_copy(zf_hbm, acc.at[pl.ds(sid * NPS, NPS)])
        if with_deg:
            pltpu.sync_copy(zd_hbm, dacc.at[pl.ds(sid * NPS, NPS)])
            pltpu.sync_copy(e1_hbm, ones_v)
        plsc.subcore_barrier()

        def prep(j, s):
            pltpu.sync_copy(idx_hbm.at[pl.ds((wid * CPW + s) * 2, 2)],
                            idx_v.at[pl.ds(2 * j, 2)])
            pltpu.async_copy(h_hbm.at[idx_v.at[2 * j + 1]],
                             rows.at[pl.ds(j * CH, CH)], sem_g[j])

        def wait_gather(j):
            pltpu.make_async_copy(h_hbm.at[idx_v.at[2 * j + 1]],
                                  rows.at[pl.ds(j * CH, CH)],
                                  sem_g[j]).wait()

        def fire(j):
            pltpu.async_copy(rows.at[pl.ds(j * CH, CH)],
                             acc.at[idx_v.at[2 * j]], sem_s[j], add=True)
            if with_deg:
                pltpu.async_copy(ones_v, dacc.at[idx_v.at[2 * j]], sem_d[j],
                                 add=True)

        def wait_scatter(j):
            pltpu.make_async_copy(rows.at[pl.ds(j * CH, CH)],
                                  acc.at[idx_v.at[2 * j]], sem_s[j]).wait()
            if with_deg:
                pltpu.make_async_copy(ones_v, dacc.at[idx_v.at[2 * j]],
                                      sem_d[j]).wait()

        def slot(j, s, do_wait_scatter=True, do_prep=True):
            jp = (j + 2) % NQ
            if do_wait_scatter:
                wait_scatter(jp)
            if do_prep:
                prep(jp, s + 2)
            wait_gather(j)
            fire(j)

        def steady(i, carry):
            for j in range(NQ):
                slot(j, i * NQ + j)
            return carry

        prep(0, 0)
        prep(1, 1)
        slot(0, 0, do_wait_scatter=False)
        slot(1, 1, do_wait_scatter=False)
        slot(2, 2)
        slot(3, 3)
        lax.fori_loop(1, OUTER - 1, steady, 0)
        for j in range(NQ):
            s = (OUTER - 1) * NQ + j
            slot(j, s, do_prep=(j < 2))
        wait_scatter(2)
        wait_scatter(3)
        plsc.subcore_barrier()

        sl = pl.ds(sid * NPS, NPS)
        pltpu.sync_copy(acc.at[sl], p_hbm.at[cid].at[sl])
        if with_deg:
            pltpu.sync_copy(dacc.at[sl], dp_hbm.at[cid].at[sl])

    return pl.kernel(body, out_type=tuple(out_type), mesh=mesh,
                     scratch_types=tuple(scratch),
                     compiler_params=pltpu.CompilerParams(
                         use_tc_tiling_on_sc=False))


def _edge_chunks(edge_index, N):
    E = edge_index.shape[1]
    Nz = ((N + NS * 8 - 1) // (NS * 8)) * (NS * 8)
    unit = NW * CH * NQ
    Ep = ((E + unit - 1) // unit) * unit
    if Ep != E:
        pad = jnp.stack([jnp.full((Ep - E,), Nz, jnp.int32),
                         jnp.zeros((Ep - E,), jnp.int32)])
        edge_index = jnp.concatenate([edge_index, pad], axis=1)
    chunks = edge_index.reshape(2, Ep // CH, CH).transpose(1, 0, 2)
    return chunks.reshape(Ep // CH * 2, CH), Ep


def _sc_scatter(idx_chunks, Ep, h, with_deg):
    N, F = h.shape
    Nz = ((N + NS * 8 - 1) // (NS * 8)) * (NS * 8)
    NPS = Nz // NS
    zf = jnp.zeros((NPS, F), jnp.float32)
    zd = jnp.zeros((NPS, DW), jnp.float32)
    e1 = jnp.zeros((CH, DW), jnp.float32).at[:, 0].set(1.0)
    k = _make_sc_scatter(N, F, Ep, with_deg)
    return k(idx_chunks, h, zf, zd, e1)


def _mm_body(x_ref, w_ref, o_ref):
    o_ref[...] = jnp.dot(x_ref[...], w_ref[...],
                         preferred_element_type=jnp.float32)


def _mid_body(p_ref, h1_ref, dp_ref, b1_ref, w2_ref, o_ref):
    deg = dp_ref[0, :, 0:1] + dp_ref[1, :, 0:1] + 1.0
    s = (p_ref[0] + p_ref[1] + h1_ref[...]) * (1.0 / deg) + b1_ref[...]
    h = jnp.maximum(s, 0.0)
    o_ref[...] = jnp.dot(h, w2_ref[...], preferred_element_type=jnp.float32)


def _out_body(q_ref, h2_ref, dp_ref, b2_ref, o_ref):
    deg = dp_ref[0, :, 0:1] + dp_ref[1, :, 0:1] + 1.0
    o_ref[...] = (q_ref[0] + q_ref[1] + h2_ref[...]) * (1.0 / deg) \
        + b2_ref[...]


def _row_block(N):
    for r in (1000, 500, 250, 200, 125, 100, 50, 40, 25, 20, 10, 8, 5, 4, 2):
        if N % r == 0:
            return r
    return N


def kernel(x, edge_index, W1, b1, W2, b2):
    N, NF = x.shape
    NH = W1.shape[1]
    F2 = W2.shape[1]
    idx_chunks, Ep = _edge_chunks(edge_index, N)
    R = _row_block(N)
    G = N // R

    h1 = pl.pallas_call(
        _mm_body,
        grid=(G,),
        in_specs=[pl.BlockSpec((R, NF), lambda i: (i, 0)),
                  pl.BlockSpec((NF, NH), lambda i: (0, 0))],
        out_specs=pl.BlockSpec((R, NH), lambda i: (i, 0)),
        out_shape=jax.ShapeDtypeStruct((N, NH), jnp.float32),
    )(x, W1)

    p1, dp = _sc_scatter(idx_chunks, Ep, h1, with_deg=True)

    h2 = pl.pallas_call(
        _mid_body,
        grid=(G,),
        in_specs=[pl.BlockSpec((NC, R, NH), lambda i: (0, i, 0)),
                  pl.BlockSpec((R, NH), lambda i: (i, 0)),
                  pl.BlockSpec((NC, R, DW), lambda i: (0, i, 0)),
                  pl.BlockSpec((1, NH), lambda i: (0, 0)),
                  pl.BlockSpec((NH, F2), lambda i: (0, 0))],
        out_specs=pl.BlockSpec((R, F2), lambda i: (i, 0)),
        out_shape=jax.ShapeDtypeStruct((N, F2), jnp.float32),
    )(p1[:, :N], h1, dp[:, :N], b1.reshape(1, NH), W2)

    (p2,) = _sc_scatter(idx_chunks, Ep, h2, with_deg=False)

    out = pl.pallas_call(
        _out_body,
        grid=(G,),
        in_specs=[pl.BlockSpec((NC, R, F2), lambda i: (0, i, 0)),
                  pl.BlockSpec((R, F2), lambda i: (i, 0)),
                  pl.BlockSpec((NC, R, DW), lambda i: (0, i, 0)),
                  pl.BlockSpec((1, F2), lambda i: (0, 0))],
        out_specs=pl.BlockSpec((R, F2), lambda i: (i, 0)),
        out_shape=jax.ShapeDtypeStruct((N, F2), jnp.float32),
    )(p2[:, :N], h2, dp[:, :N], b2.reshape(1, F2))

    return out


# speedup vs baseline: 3.1040x; 2.7376x over previous
import functools

import jax
import jax.numpy as jnp
from jax import lax
from jax.experimental import pallas as pl
from jax.experimental.pallas import tpu as pltpu
from jax.experimental.pallas import tpu_sc as plsc

NC = 2
NS = 16
NW = NC * NS
CH = 128
NQ = 4
DW = 16


F = 64
WO = 128


@functools.lru_cache(maxsize=None)
def _make_sc_scatter(N, Ep, passes):
    CPW = Ep // (NW * CH)
    OUTER = CPW // NQ
    Nz = ((N + NS * 8 - 1) // (NS * 8)) * (NS * 8)
    NPS = Nz // NS
    Nacc = Nz + 16
    any_deg = any(wd for _, wd in passes)

    mesh = plsc.VectorSubcoreMesh(
        core_axis_name="c", subcore_axis_name="s", num_cores=NC, num_subcores=NS
    )

    out_type = [jax.ShapeDtypeStruct((NC, Nz, WO), jnp.float32)]
    scratch = [
        pltpu.VMEM((NQ * 2, CH), jnp.int32),
        pltpu.VMEM((NQ * CH, F), jnp.float32),
        pltpu.VMEM_SHARED((Nz, F), jnp.float32),
        pltpu.VMEM_SHARED((Nacc, F), jnp.float32),
        [pltpu.SemaphoreType.DMA for _ in range(NQ)],
        [pltpu.SemaphoreType.DMA for _ in range(NQ)],
        [pltpu.SemaphoreType.DMA for _ in range(NQ)],
    ]
    if any_deg:
        out_type.append(jax.ShapeDtypeStruct((NC, Nz, DW), jnp.float32))
        scratch += [
            pltpu.VMEM((CH, DW), jnp.float32),
            pltpu.VMEM_SHARED((Nacc, DW), jnp.float32),
            [pltpu.SemaphoreType.DMA for _ in range(NQ)],
        ]

    def body(idx_hbm, h_hbm, zf_hbm, zd_hbm, e1_hbm, p_hbm, *rest):
        if any_deg:
            (dp_hbm, idx_v, rows, hs, acc, sem_i, sem_g, sem_s,
             ones_v, dacc, sem_d) = rest
        else:
            idx_v, rows, hs, acc, sem_i, sem_g, sem_s = rest
        cid = lax.axis_index("c")
        sid = lax.axis_index("s")
        wid = cid * NS + sid
        sl0 = pl.ds(sid * NPS, NPS)

        def start_idx(j, s):
            off = pl.ds((wid * CPW + s) * CH, CH)
            pltpu.async_copy(idx_hbm.at[0, off], idx_v.at[2 * j], sem_i[j])
            pltpu.async_copy(idx_hbm.at[1, off], idx_v.at[2 * j + 1],
                             sem_i[j])

        def wait_idx(j, s):
            off = pl.ds((wid * CPW + s) * CH, CH)
            pltpu.make_async_copy(idx_hbm.at[0, off], idx_v.at[2 * j],
                                  sem_i[j]).wait()
            pltpu.make_async_copy(idx_hbm.at[1, off], idx_v.at[2 * j + 1],
                                  sem_i[j]).wait()

        def start_gather(j):
            pltpu.async_copy(hs.at[idx_v.at[2 * j + 1]],
                             rows.at[pl.ds(j * CH, CH)], sem_g[j])

        def wait_gather(j):
            pltpu.make_async_copy(hs.at[idx_v.at[2 * j + 1]],
                                  rows.at[pl.ds(j * CH, CH)],
                                  sem_g[j]).wait()

        def run_pass(coff, wd):
            pltpu.sync_copy(h_hbm.at[sl0, pl.ds(coff, F)], hs.at[sl0])
            pltpu.sync_copy(zf_hbm, acc.at[sl0])
            if wd:
                pltpu.sync_copy(zd_hbm, dacc.at[sl0])
                pltpu.sync_copy(e1_hbm, ones_v)
            plsc.subcore_barrier()

            def fire(j):
                pltpu.async_copy(rows.at[pl.ds(j * CH, CH)],
                                 acc.at[idx_v.at[2 * j]], sem_s[j], add=True)
                if wd:
                    pltpu.async_copy(ones_v, dacc.at[idx_v.at[2 * j]],
                                     sem_d[j], add=True)

            def wait_scatter(j):
                pltpu.make_async_copy(rows.at[pl.ds(j * CH, CH)],
                                      acc.at[idx_v.at[2 * j]],
                                      sem_s[j]).wait()
                if wd:
                    pltpu.make_async_copy(ones_v, dacc.at[idx_v.at[2 * j]],
                                          sem_d[j]).wait()

            def slot(j, s, w_sc=True, p_idx=True, p_gat=True):
                jp2 = (j + 2) % NQ
                jp1 = (j + 1) % NQ
                if w_sc:
                    wait_scatter(jp2)
                if p_idx:
                    start_idx(jp2, s + 2)
                if p_gat:
                    wait_idx(jp1, s + 1)
                    start_gather(jp1)
                wait_gather(j)
                fire(j)

            def steady(i, carry):
                for j in range(NQ):
                    slot(j, i * NQ + j)
                return carry

            start_idx(0, 0)
            start_idx(1, 1)
            wait_idx(0, 0)
            start_gather(0)
            slot(0, 0, w_sc=False)
            slot(1, 1, w_sc=False)
            slot(2, 2)
            slot(3, 3)
            lax.fori_loop(1, OUTER - 1, steady, 0)
            for j in range(NQ):
                s = (OUTER - 1) * NQ + j
                slot(j, s, p_idx=(j < 2), p_gat=(j < NQ - 1))
            wait_scatter(2)
            wait_scatter(3)
            plsc.subcore_barrier()

            pltpu.sync_copy(acc.at[sl0],
                            p_hbm.at[cid].at[sl0, pl.ds(coff, F)])
            if wd:
                pltpu.sync_copy(dacc.at[sl0], dp_hbm.at[cid].at[sl0])

        for coff, wd in passes:
            run_pass(coff, wd)

    return pl.kernel(body, out_type=tuple(out_type), mesh=mesh,
                     scratch_types=tuple(scratch),
                     compiler_params=pltpu.CompilerParams(
                         use_tc_tiling_on_sc=False))


def _edge_chunks(edge_index, N):
    E = edge_index.shape[1]
    Nz = ((N + NS * 8 - 1) // (NS * 8)) * (NS * 8)
    if E % NW:
        extra = NW - E % NW
        pad = jnp.stack([Nz + (jnp.arange(extra, dtype=jnp.int32) % 16),
                         jnp.zeros((extra,), jnp.int32)])
        edge_index = jnp.concatenate([edge_index, pad], axis=1)
        E += extra
    EW = E // NW
    unit = CH * NQ
    EWp = ((EW + unit - 1) // unit) * unit
    ei = edge_index.reshape(2, NW, EW)
    if EWp != EW:
        padrow = Nz + (jnp.arange(EWp - EW, dtype=jnp.int32) % 16)
        pad = jnp.stack([jnp.broadcast_to(padrow, (NW, EWp - EW)),
                         jnp.zeros((NW, EWp - EW), jnp.int32)])
        ei = jnp.concatenate([ei, pad], axis=2)
    Ep = NW * EWp
    return ei.reshape(2, Ep), Ep


def _sc_scatter(idx_chunks, Ep, h, N, passes):
    Nz = ((N + NS * 8 - 1) // (NS * 8)) * (NS * 8)
    NPS = Nz // NS
    zf = jnp.zeros((NPS, F), jnp.float32)
    zd = jnp.zeros((NPS, DW), jnp.float32)
    e1 = jnp.zeros((CH, DW), jnp.float32).at[:, 0].set(1.0)
    k = _make_sc_scatter(N, Ep, passes)
    return k(idx_chunks, h, zf, zd, e1)


def _mm_body(x_ref, w_ref, o_ref):
    o_ref[...] = jnp.dot(x_ref[...], w_ref[...],
                         preferred_element_type=jnp.float32)


def _mid_body(p_ref, h1_ref, dp_ref, b1_ref, w2_ref, o_ref):
    deg = dp_ref[0, :, 0:1] + dp_ref[1, :, 0:1] + 1.0
    s = (p_ref[0] + p_ref[1] + h1_ref[...]) * (1.0 / deg) + b1_ref[...]
    h = jnp.maximum(s, 0.0)
    h2 = jnp.dot(h, w2_ref[...], preferred_element_type=jnp.float32)
    o_ref[...] = jnp.concatenate([h2, h2], axis=1)


def _out_body(q_ref, h2_ref, dp_ref, b2_ref, o_ref):
    F2 = o_ref.shape[1]
    deg = dp_ref[0, :, 0:1] + dp_ref[1, :, 0:1] + 1.0
    o_ref[...] = (q_ref[0, :, :F2] + q_ref[1, :, :F2] + h2_ref[:, :F2]) \
        * (1.0 / deg) + b2_ref[...]


def _row_block(N):
    for r in (1000, 500, 250, 200, 125, 100, 50, 40, 25, 20, 10, 8, 5, 4, 2):
        if N % r == 0:
            return r
    return N


def kernel(x, edge_index, W1, b1, W2, b2):
    N, NF = x.shape
    NH = W1.shape[1]
    F2 = W2.shape[1]
    idx_chunks, Ep = _edge_chunks(edge_index, N)
    Nz = ((N + NS * 8 - 1) // (NS * 8)) * (NS * 8)
    Rz = Nz // NS
    Gz = NS
    R = _row_block(N)
    G = N // R

    h1 = pl.pallas_call(
        _mm_body,
        grid=(Gz,),
        in_specs=[pl.BlockSpec((Rz, NF), lambda i: (i, 0)),
                  pl.BlockSpec((NF, NH), lambda i: (0, 0))],
        out_specs=pl.BlockSpec((Rz, NH), lambda i: (i, 0)),
        out_shape=jax.ShapeDtypeStruct((Nz, NH), jnp.float32),
    )(x, W1)

    p1, dp = _sc_scatter(idx_chunks, Ep, h1, N, ((0, True), (NH // 2, False)))

    h2 = pl.pallas_call(
        _mid_body,
        grid=(Gz,),
        in_specs=[pl.BlockSpec((NC, Rz, WO), lambda i: (0, i, 0)),
                  pl.BlockSpec((Rz, NH), lambda i: (i, 0)),
                  pl.BlockSpec((NC, Rz, DW), lambda i: (0, i, 0)),
                  pl.BlockSpec((1, NH), lambda i: (0, 0)),
                  pl.BlockSpec((NH, F2), lambda i: (0, 0))],
        out_specs=pl.BlockSpec((Rz, 2 * F2), lambda i: (i, 0)),
        out_shape=jax.ShapeDtypeStruct((Nz, 2 * F2), jnp.float32),
    )(p1, h1, dp, b1.reshape(1, NH), W2)

    (p2,) = _sc_scatter(idx_chunks, Ep, h2, N, ((0, False),))

    out = pl.pallas_call(
        _out_body,
        grid=(G,),
        in_specs=[pl.BlockSpec((NC, R, WO), lambda i: (0, i, 0)),
                  pl.BlockSpec((R, 2 * F2), lambda i: (i, 0)),
                  pl.BlockSpec((NC, R, DW), lambda i: (0, i, 0)),
                  pl.BlockSpec((1, F2), lambda i: (0, 0))],
        out_specs=pl.BlockSpec((R, F2), lambda i: (i, 0)),
        out_shape=jax.ShapeDtypeStruct((N, F2), jnp.float32),
    )(p2, h2, dp, b2.reshape(1, F2))

    return out
